# baseline (speedup 1.0000x reference)
import jax
import jax.numpy as jnp
from jax.experimental import pallas as pl
from jax.experimental.pallas import tpu as pltpu


def _tc_kernel(B, F, h, w, n_rep, interpret=False):
    hw = h * w
    assert B % n_rep == 0

    def body(row_ref, col_ref, out_ref, pos_ref, sem):
        colT = col_ref[...].T
        rowT = row_ref[...].T
        top = jnp.tile(colT, (1, h))
        bot = jnp.repeat(rowT, w, axis=1)
        pos = jnp.concatenate([top, bot], axis=0)
        for r in range(n_rep):
            pos_ref[r] = pos

        n_dma = B // n_rep
        for i in range(n_dma):
            pltpu.make_async_copy(
                pos_ref, out_ref.at[pl.ds(i * n_rep, n_rep)], sem
            ).start()
        for i in range(n_dma):
            pltpu.make_async_copy(
                pos_ref, out_ref.at[pl.ds(i * n_rep, n_rep)], sem
            ).wait()

    return pl.pallas_call(
        body,
        in_specs=[
            pl.BlockSpec((h, F), lambda: (0, 0)),
            pl.BlockSpec((w, F), lambda: (0, 0)),
        ],
        out_specs=pl.BlockSpec(memory_space=pl.ANY),
        out_shape=jax.ShapeDtypeStruct((B, 2 * F, hw), jnp.float32),
        scratch_shapes=[
            pltpu.VMEM((n_rep, 2 * F, hw), jnp.float32),
            pltpu.SemaphoreType.DMA,
        ],
        interpret=interpret,
    )


def kernel(token_tensors, row_embed_w, col_embed_w):
    B, _, h, w = token_tensors.shape
    F = row_embed_w.shape[1]
    out = _tc_kernel(B, F, h, w, n_rep=4)(row_embed_w, col_embed_w)
    return out.reshape(B, 2 * F, h, w)


# speedup vs baseline: 1.0037x; 1.0037x over previous
import jax
import jax.numpy as jnp
from jax.experimental import pallas as pl
from jax.experimental.pallas import tpu as pltpu


def _tc_kernel(B, F, h, w, n_rep, interpret=False):
    hw = h * w
    assert B % n_rep == 0

    n_sem = 8

    def body(row_ref, col_ref, out_ref, pos_ref, sems):
        colT = col_ref[...].T
        rowT = row_ref[...].T
        top = jnp.tile(colT, (1, h))
        bot = jnp.repeat(rowT, w, axis=1)
        pos = jnp.concatenate([top, bot], axis=0)
        for r in range(n_rep):
            pos_ref[r] = pos

        n_dma = B // n_rep
        for i in range(n_dma):
            pltpu.make_async_copy(
                pos_ref, out_ref.at[pl.ds(i * n_rep, n_rep)], sems.at[i % n_sem]
            ).start()
        for i in range(n_dma):
            pltpu.make_async_copy(
                pos_ref, out_ref.at[pl.ds(i * n_rep, n_rep)], sems.at[i % n_sem]
            ).wait()

    return pl.pallas_call(
        body,
        in_specs=[
            pl.BlockSpec((h, F), lambda: (0, 0)),
            pl.BlockSpec((w, F), lambda: (0, 0)),
        ],
        out_specs=pl.BlockSpec(memory_space=pl.ANY),
        out_shape=jax.ShapeDtypeStruct((B, 2 * F, hw), jnp.float32),
        scratch_shapes=[
            pltpu.VMEM((n_rep, 2 * F, hw), jnp.float32),
            pltpu.SemaphoreType.DMA((n_sem,)),
        ],
        interpret=interpret,
    )


def kernel(token_tensors, row_embed_w, col_embed_w):
    B, _, h, w = token_tensors.shape
    F = row_embed_w.shape[1]
    out = _tc_kernel(B, F, h, w, n_rep=4)(row_embed_w, col_embed_w)
    return out.reshape(B, 2 * F, h, w)
